# baseline (speedup 1.0000x reference)
import functools

import jax
import jax.numpy as jnp
from jax import lax
from jax.experimental import pallas as pl
from jax.experimental.pallas import tpu as pltpu
from jax.experimental.pallas import tpu_sc as plsc

N = 10000
E = 160000
G = 64
CH = 256

NHALF = 2
EH = E // NHALF

BE = 1600
BN = 1000

F32 = jnp.float32


def _silu(x):
    return x * jax.nn.sigmoid(x)


def _mlp2_body(x_ref, w1_ref, b1_ref, w2_ref, b2_ref, o_ref, *, outer_silu):
    h = _silu(jnp.dot(x_ref[...], w1_ref[...], preferred_element_type=F32)
              + b1_ref[...])
    o = jnp.dot(h, w2_ref[...], preferred_element_type=F32) + b2_ref[...]
    o_ref[...] = _silu(o) if outer_silu else o


def _mlp2(x, p0, p1, *, block, nb, off=0, outer_silu=False):
    din = x.shape[1]
    dout = p1["w"].shape[1]
    b1 = p0["b"].reshape(1, -1)
    b2 = p1["b"].reshape(1, -1)
    return pl.pallas_call(
        functools.partial(_mlp2_body, outer_silu=outer_silu),
        grid=(nb,),
        in_specs=[
            pl.BlockSpec((block, din), lambda i: (off + i, 0)),
            pl.BlockSpec(p0["w"].shape, lambda i: (0, 0)),
            pl.BlockSpec(b1.shape, lambda i: (0, 0)),
            pl.BlockSpec(p1["w"].shape, lambda i: (0, 0)),
            pl.BlockSpec(b2.shape, lambda i: (0, 0)),
        ],
        out_specs=pl.BlockSpec((block, dout), lambda i: (i, 0)),
        out_shape=jax.ShapeDtypeStruct((nb * block, dout), F32),
    )(x, p0["w"], b1, p1["w"], b2)


def _edge_layer_body(gs_ref, gr_ref, xe_ref, we_ref, b1_ref, w2_ref, b2_ref,
                     o_ref):
    a = (gs_ref[...] + gr_ref[...]
         + jnp.dot(xe_ref[...], we_ref[...], preferred_element_type=F32)
         + b1_ref[...])
    h = _silu(a)
    o = jnp.dot(h, w2_ref[...], preferred_element_type=F32) + b2_ref[...]
    o_ref[...] = _silu(o)


def _edge_layer(gath, xe_h, we, b1, w2, b2):
    nb = EH // BE
    b1 = b1.reshape(1, -1)
    b2 = b2.reshape(1, -1)
    return pl.pallas_call(
        _edge_layer_body,
        grid=(nb,),
        in_specs=[
            pl.BlockSpec((BE, CH), lambda i: (i, 0)),
            pl.BlockSpec((BE, CH), lambda i: (nb + i, 0)),
            pl.BlockSpec((BE, CH), lambda i: (i, 0)),
            pl.BlockSpec((CH, CH), lambda i: (0, 0)),
            pl.BlockSpec((1, CH), lambda i: (0, 0)),
            pl.BlockSpec((CH, CH), lambda i: (0, 0)),
            pl.BlockSpec((1, CH), lambda i: (0, 0)),
        ],
        out_specs=pl.BlockSpec((BE, CH), lambda i: (i, 0)),
        out_shape=jax.ShapeDtypeStruct((EH, CH), F32),
    )(gath, gath, xe_h, we, b1, w2, b2)


def _node_layer_body(xn_ref, a0_ref, a1_ref, wx_ref, wa_ref, b1_ref, w2_ref,
                     b2_ref, o_ref):
    ag = a0_ref[...] + a1_ref[...]
    a = (jnp.dot(xn_ref[...], wx_ref[...], preferred_element_type=F32)
         + jnp.dot(ag, wa_ref[...], preferred_element_type=F32)
         + b1_ref[...])
    h = _silu(a)
    o_ref[...] = jnp.dot(h, w2_ref[...], preferred_element_type=F32) + b2_ref[...]


def _node_layer(xn, ag0, ag1, wx, wa, b1, w2, b2):
    nb = N // BN
    b1 = b1.reshape(1, -1)
    b2 = b2.reshape(1, -1)
    return pl.pallas_call(
        _node_layer_body,
        grid=(nb,),
        in_specs=[
            pl.BlockSpec((BN, CH), lambda i: (i, 0)),
            pl.BlockSpec((BN, CH), lambda i: (i, 0)),
            pl.BlockSpec((BN, CH), lambda i: (i, 0)),
            pl.BlockSpec((CH, CH), lambda i: (0, 0)),
            pl.BlockSpec((CH, CH), lambda i: (0, 0)),
            pl.BlockSpec((1, CH), lambda i: (0, 0)),
            pl.BlockSpec((CH, CH), lambda i: (0, 0)),
            pl.BlockSpec((1, CH), lambda i: (0, 0)),
        ],
        out_specs=pl.BlockSpec((BN, CH), lambda i: (i, 0)),
        out_shape=jax.ShapeDtypeStruct((N, CH), F32),
    )(xn, ag0, ag1, wx, wa, b1, w2, b2)


def _pq_body(xn_ref, w_ref, o_ref):
    o_ref[...] = jnp.dot(xn_ref[...], w_ref[0], preferred_element_type=F32)


def _pq(xn, wsr):
    nb = N // BN
    return pl.pallas_call(
        _pq_body,
        grid=(2, nb),
        in_specs=[
            pl.BlockSpec((BN, CH), lambda c, i: (i, 0)),
            pl.BlockSpec((1, CH, CH), lambda c, i: (c, 0, 0)),
        ],
        out_specs=pl.BlockSpec((BN, CH), lambda c, i: (c * nb + i, 0)),
        out_shape=jax.ShapeDtypeStruct((2 * N, CH), F32),
    )(xn, wsr)


def _global_body(xn_ref, b_ref, gw1_ref, gb1_ref, gw2_ref, gb2_ref, o_ref,
                 acc_ref):
    i = pl.program_id(0)

    @pl.when(i == 0)
    def _init():
        acc_ref[...] = jnp.zeros_like(acc_ref)

    seg = b_ref[...]
    gids = lax.broadcasted_iota(jnp.int32, (1, G), 1)
    onehot = (seg == gids).astype(F32)
    acc_ref[...] += lax.dot_general(
        onehot, xn_ref[...], (((0,), (0,)), ((), ())),
        preferred_element_type=F32)

    @pl.when(i == pl.num_programs(0) - 1)
    def _fin():
        xg = acc_ref[...]
        h = _silu(jnp.dot(xg, gw1_ref[...], preferred_element_type=F32)
                  + gb1_ref[...])
        o_ref[...] = jnp.dot(h, gw2_ref[...], preferred_element_type=F32) \
            + gb2_ref[...]


def _global_readout(xn_out, batch, p0, p1):
    nb = N // BN
    b2d = batch.reshape(N, 1)
    gb1 = p0["b"].reshape(1, -1)
    gb2 = p1["b"].reshape(1, -1)
    return pl.pallas_call(
        _global_body,
        grid=(nb,),
        in_specs=[
            pl.BlockSpec((BN, CH), lambda i: (i, 0)),
            pl.BlockSpec((BN, 1), lambda i: (i, 0)),
            pl.BlockSpec(p0["w"].shape, lambda i: (0, 0)),
            pl.BlockSpec(gb1.shape, lambda i: (0, 0)),
            pl.BlockSpec(p1["w"].shape, lambda i: (0, 0)),
            pl.BlockSpec(gb2.shape, lambda i: (0, 0)),
        ],
        out_specs=pl.BlockSpec((G, G), lambda i: (0, 0)),
        out_shape=jax.ShapeDtypeStruct((G, G), F32),
        scratch_shapes=[pltpu.VMEM((G, CH), F32)],
    )(xn_out, b2d, p0["w"], gb1, p1["w"], gb2)


NC, NS = 2, 16
NW = NC * NS

GCHUNK = 40
G_PER_W = 2 * EH // NW
G_NCH = G_PER_W // GCHUNK

S_PER_W = EH // NS
S_NCH = S_PER_W // GCHUNK
CHH = CH // 2


def _gather_rows(table, idx_r):
    mesh = plsc.VectorSubcoreMesh(core_axis_name="c", subcore_axis_name="s")

    @functools.partial(
        pl.kernel,
        out_type=jax.ShapeDtypeStruct((2 * EH, CH), F32),
        mesh=mesh,
        scratch_types=[
            pltpu.VMEM((G_NCH, GCHUNK), jnp.int32),
            pltpu.VMEM((GCHUNK, CH), F32),
            pltpu.VMEM((GCHUNK, CH), F32),
            pltpu.SemaphoreType.DMA,
            pltpu.SemaphoreType.DMA,
        ],
    )
    def k(table_hbm, idx_hbm, out_hbm, idx_v, buf0, buf1, sem0, sem1):
        wid = lax.axis_index("s") * NC + lax.axis_index("c")
        base = wid * G_PER_W
        pltpu.sync_copy(idx_hbm.at[wid], idx_v)
        pltpu.async_copy(table_hbm.at[idx_v.at[0]], buf0, sem0)

        def body(t, _):
            a = 2 * t

            @pl.when(a + 1 < G_NCH)
            def _l1():
                pltpu.async_copy(table_hbm.at[idx_v.at[a + 1]], buf1, sem1)

            pltpu.make_async_copy(table_hbm.at[idx_v.at[a]], buf0, sem0).wait()
            pltpu.sync_copy(buf0,
                            out_hbm.at[pl.ds(base + a * GCHUNK, GCHUNK), :])

            @pl.when(a + 2 < G_NCH)
            def _l2():
                pltpu.async_copy(table_hbm.at[idx_v.at[a + 2]], buf0, sem0)

            @pl.when(a + 1 < G_NCH)
            def _w1():
                pltpu.make_async_copy(table_hbm.at[idx_v.at[a + 1]], buf1,
                                      sem1).wait()
                pltpu.sync_copy(
                    buf1, out_hbm.at[pl.ds(base + (a + 1) * GCHUNK, GCHUNK), :])

            return 0

        lax.fori_loop(0, (G_NCH + 1) // 2, body, 0, unroll=False)

    return k(table, idx_r)


def _segment_sum(xe_h, idx_r, zeros_half):
    mesh = plsc.VectorSubcoreMesh(core_axis_name="c", subcore_axis_name="s")

    @functools.partial(
        pl.kernel,
        out_type=jax.ShapeDtypeStruct((N, CH), F32),
        mesh=mesh,
        scratch_types=[
            pltpu.VMEM((S_NCH, GCHUNK), jnp.int32),
            pltpu.VMEM((GCHUNK, CHH), F32),
            pltpu.VMEM((GCHUNK, CHH), F32),
            pltpu.VMEM_SHARED((N, CHH), F32),
            pltpu.SemaphoreType.DMA,
            pltpu.SemaphoreType.DMA,
        ],
    )
    def k(xe_hbm, idx_hbm, z_hbm, out_hbm, idx_v, buf0, buf1, acc, sem0,
          sem1):
        cid = lax.axis_index("c")
        sid = lax.axis_index("s")
        col0 = cid * CHH
        zbase = sid * 624
        pltpu.sync_copy(z_hbm.at[pl.ds(zbase, 624)], acc.at[pl.ds(zbase, 624)])

        @pl.when(sid == NS - 1)
        def _tail():
            pltpu.sync_copy(z_hbm.at[pl.ds(9984, 16)], acc.at[pl.ds(9984, 16)])

        plsc.subcore_barrier()

        ebase = sid * S_PER_W
        pltpu.sync_copy(idx_hbm.at[sid], idx_v)

        def _src(j):
            return xe_hbm.at[pl.ds(ebase + j * GCHUNK, GCHUNK),
                             pl.ds(col0, CHH)]

        pltpu.async_copy(_src(0), buf0, sem0)

        def body(t, _):
            a = 2 * t

            @pl.when(a + 1 < S_NCH)
            def _l1():
                pltpu.async_copy(_src(a + 1), buf1, sem1)

            pltpu.make_async_copy(_src(a), buf0, sem0).wait()
            pltpu.sync_copy(buf0, acc.at[idx_v.at[a]], add=True)

            @pl.when(a + 2 < S_NCH)
            def _l2():
                pltpu.async_copy(_src(a + 2), buf0, sem0)

            @pl.when(a + 1 < S_NCH)
            def _w1():
                pltpu.make_async_copy(_src(a + 1), buf1, sem1).wait()
                pltpu.sync_copy(buf1, acc.at[idx_v.at[a + 1]], add=True)

            return 0

        lax.fori_loop(0, (S_NCH + 1) // 2, body, 0, unroll=False)
        plsc.subcore_barrier()
        pltpu.sync_copy(acc.at[pl.ds(zbase, 624)],
                        out_hbm.at[pl.ds(zbase, 624), pl.ds(col0, CHH)])

        @pl.when(sid == NS - 1)
        def _tail2():
            pltpu.sync_copy(acc.at[pl.ds(9984, 16)],
                            out_hbm.at[pl.ds(9984, 16), pl.ds(col0, CHH)])

    return k(xe_h, idx_r, zeros_half)


def kernel(x_nodes, x_edges, params, edge_index, batch, pbc):
    sender = edge_index[0]
    receiver = edge_index[1]

    idx2 = []
    recv_r = []
    for h in range(NHALF):
        snd_h = lax.dynamic_slice_in_dim(sender, h * EH, EH)
        rcv_h = lax.dynamic_slice_in_dim(receiver, h * EH, EH)
        idx2.append(jnp.concatenate([snd_h, rcv_h + N])
                    .reshape(NW, G_NCH, GCHUNK))
        recv_r.append(rcv_h.reshape(NS, S_NCH, GCHUNK))
    zeros_half = jnp.zeros((N, CHH), F32)

    nbh = EH // BE
    xe = [_mlp2(x_edges, *params["embed_edges"], block=BE, nb=nbh,
                off=h * nbh) for h in range(NHALF)]
    xn = _mlp2(x_nodes, *params["embed_nodes"], block=BN, nb=N // BN)

    for lp in params["layers"]:
        w1 = lp["edge"][0]["w"]
        wsr = jnp.stack([w1[:CH], w1[CH:2 * CH]])
        we = w1[2 * CH:]
        b1 = lp["edge"][0]["b"]
        w2, b2 = lp["edge"][1]["w"], lp["edge"][1]["b"]

        T = _pq(xn, wsr)
        gath = [_gather_rows(T, idx2[h]) for h in range(NHALF)]
        xe = [_edge_layer(gath[h], xe[h], we, b1, w2, b2)
              for h in range(NHALF)]
        aggr = [_segment_sum(xe[h], recv_r[h], zeros_half)
                for h in range(NHALF)]

        nw1 = lp["node"][0]["w"]
        xn = _node_layer(xn, aggr[0], aggr[1], nw1[:CH], nw1[CH:],
                         lp["node"][0]["b"], lp["node"][1]["w"],
                         lp["node"][1]["b"])

    xn_out = _mlp2(xn, *params["node_readout"], block=BN, nb=N // BN)
    xe_out = jnp.concatenate(
        [_mlp2(xe[h], *params["edge_readout"], block=BE, nb=nbh)
         for h in range(NHALF)])
    xg = _global_readout(xn_out, batch, *params["global_readout"])
    return (xn_out, xe_out, xg)


# speedup vs baseline: 1.2322x; 1.2322x over previous
import functools

import jax
import jax.numpy as jnp
from jax import lax
from jax.experimental import pallas as pl
from jax.experimental.pallas import tpu as pltpu
from jax.experimental.pallas import tpu_sc as plsc

N = 10000
E = 160000
G = 64
CH = 256

NHALF = 2
EH = E // NHALF

BE = 1600
BN = 1000

F32 = jnp.float32
F16 = jnp.bfloat16


def _silu(x):
    return x * jax.nn.sigmoid(x)


def _mlp2_body(x_ref, w1_ref, b1_ref, w2_ref, b2_ref, o_ref, *, outer_silu):
    h = _silu(jnp.dot(x_ref[...], w1_ref[...], preferred_element_type=F32)
              + b1_ref[...])
    o = jnp.dot(h, w2_ref[...], preferred_element_type=F32) + b2_ref[...]
    o_ref[...] = _silu(o) if outer_silu else o


def _mlp2(x, p0, p1, *, block, nb, off=0, outer_silu=False):
    din = x.shape[1]
    dout = p1["w"].shape[1]
    b1 = p0["b"].reshape(1, -1)
    b2 = p1["b"].reshape(1, -1)
    return pl.pallas_call(
        functools.partial(_mlp2_body, outer_silu=outer_silu),
        grid=(nb,),
        in_specs=[
            pl.BlockSpec((block, din), lambda i: (off + i, 0)),
            pl.BlockSpec(p0["w"].shape, lambda i: (0, 0)),
            pl.BlockSpec(b1.shape, lambda i: (0, 0)),
            pl.BlockSpec(p1["w"].shape, lambda i: (0, 0)),
            pl.BlockSpec(b2.shape, lambda i: (0, 0)),
        ],
        out_specs=pl.BlockSpec((block, dout), lambda i: (i, 0)),
        out_shape=jax.ShapeDtypeStruct((nb * block, dout), F32),
    )(x, p0["w"], b1, p1["w"], b2)


def _unpack2(w):
    lo = lax.bitcast_convert_type(
        (w & 0xFFFF).astype(jnp.uint16), F16).astype(F32)
    hi = lax.bitcast_convert_type(
        (jnp.right_shift(w, 16) & 0xFFFF).astype(jnp.uint16), F16).astype(F32)
    return lo, hi


def _edge_layer_body(gs_ref, gr_ref, xe_ref, we_ref, b1_ref, w2_ref, b2_ref,
                     o_ref):
    m = jnp.dot(xe_ref[...], we_ref[...], preferred_element_type=F32)
    b1 = b1_ref[...]
    gs0, gs1 = _unpack2(gs_ref[...])
    gr0, gr1 = _unpack2(gr_ref[...])
    h0 = _silu(gs0 + gr0 + m[:, :CHH] + b1[:, :CHH])
    h1 = _silu(gs1 + gr1 + m[:, CHH:] + b1[:, CHH:])
    w2 = w2_ref[...]
    o = (jnp.dot(h0, w2[:CHH, :], preferred_element_type=F32)
         + jnp.dot(h1, w2[CHH:, :], preferred_element_type=F32)
         + b2_ref[...])
    o_ref[...] = _silu(o)


def _edge_layer(gath, xe_h, we, b1, w2, b2):
    nb = EH // BE
    b1 = b1.reshape(1, -1)
    b2 = b2.reshape(1, -1)
    return pl.pallas_call(
        _edge_layer_body,
        grid=(nb,),
        in_specs=[
            pl.BlockSpec((BE, CHH), lambda i: (i, 0)),
            pl.BlockSpec((BE, CHH), lambda i: (nb + i, 0)),
            pl.BlockSpec((BE, CH), lambda i: (i, 0)),
            pl.BlockSpec((CH, CH), lambda i: (0, 0)),
            pl.BlockSpec((1, CH), lambda i: (0, 0)),
            pl.BlockSpec((CH, CH), lambda i: (0, 0)),
            pl.BlockSpec((1, CH), lambda i: (0, 0)),
        ],
        out_specs=pl.BlockSpec((BE, CH), lambda i: (i, 0)),
        out_shape=jax.ShapeDtypeStruct((EH, CH), F32),
    )(gath, gath, xe_h, we, b1, w2, b2)


def _node_layer_body(xn_ref, a0_ref, a1_ref, wx_ref, wa_ref, b1_ref, w2_ref,
                     b2_ref, o_ref):
    ag = a0_ref[...] + a1_ref[...]
    a = (jnp.dot(xn_ref[...], wx_ref[...], preferred_element_type=F32)
         + jnp.dot(ag, wa_ref[...], preferred_element_type=F32)
         + b1_ref[...])
    h = _silu(a)
    o_ref[...] = jnp.dot(h, w2_ref[...], preferred_element_type=F32) + b2_ref[...]


def _node_layer(xn, ag0, ag1, wx, wa, b1, w2, b2):
    nb = N // BN
    b1 = b1.reshape(1, -1)
    b2 = b2.reshape(1, -1)
    return pl.pallas_call(
        _node_layer_body,
        grid=(nb,),
        in_specs=[
            pl.BlockSpec((BN, CH), lambda i: (i, 0)),
            pl.BlockSpec((BN, CH), lambda i: (i, 0)),
            pl.BlockSpec((BN, CH), lambda i: (i, 0)),
            pl.BlockSpec((CH, CH), lambda i: (0, 0)),
            pl.BlockSpec((CH, CH), lambda i: (0, 0)),
            pl.BlockSpec((1, CH), lambda i: (0, 0)),
            pl.BlockSpec((CH, CH), lambda i: (0, 0)),
            pl.BlockSpec((1, CH), lambda i: (0, 0)),
        ],
        out_specs=pl.BlockSpec((BN, CH), lambda i: (i, 0)),
        out_shape=jax.ShapeDtypeStruct((N, CH), F32),
    )(xn, ag0, ag1, wx, wa, b1, w2, b2)


def _pq_body(xn_ref, w_ref, o_ref):
    xw = jnp.dot(xn_ref[...], w_ref[0], preferred_element_type=F32)
    lo = lax.bitcast_convert_type(
        xw[:, :CHH].astype(F16), jnp.uint16).astype(jnp.int32)
    hi = lax.bitcast_convert_type(
        xw[:, CHH:].astype(F16), jnp.uint16).astype(jnp.int32)
    o_ref[...] = lo | jnp.left_shift(hi, 16)


def _pq(xn, wsr):
    nb = N // BN
    return pl.pallas_call(
        _pq_body,
        grid=(2, nb),
        in_specs=[
            pl.BlockSpec((BN, CH), lambda c, i: (i, 0)),
            pl.BlockSpec((1, CH, CH), lambda c, i: (c, 0, 0)),
        ],
        out_specs=pl.BlockSpec((BN, CHH), lambda c, i: (c * nb + i, 0)),
        out_shape=jax.ShapeDtypeStruct((2 * N, CHH), jnp.int32),
    )(xn, wsr)


def _global_body(xn_ref, b_ref, gw1_ref, gb1_ref, gw2_ref, gb2_ref, o_ref,
                 acc_ref):
    i = pl.program_id(0)

    @pl.when(i == 0)
    def _init():
        acc_ref[...] = jnp.zeros_like(acc_ref)

    seg = b_ref[...]
    gids = lax.broadcasted_iota(jnp.int32, (1, G), 1)
    onehot = (seg == gids).astype(F32)
    acc_ref[...] += lax.dot_general(
        onehot, xn_ref[...], (((0,), (0,)), ((), ())),
        preferred_element_type=F32)

    @pl.when(i == pl.num_programs(0) - 1)
    def _fin():
        xg = acc_ref[...]
        h = _silu(jnp.dot(xg, gw1_ref[...], preferred_element_type=F32)
                  + gb1_ref[...])
        o_ref[...] = jnp.dot(h, gw2_ref[...], preferred_element_type=F32) \
            + gb2_ref[...]


def _global_readout(xn_out, batch, p0, p1):
    nb = N // BN
    b2d = batch.reshape(N, 1)
    gb1 = p0["b"].reshape(1, -1)
    gb2 = p1["b"].reshape(1, -1)
    return pl.pallas_call(
        _global_body,
        grid=(nb,),
        in_specs=[
            pl.BlockSpec((BN, CH), lambda i: (i, 0)),
            pl.BlockSpec((BN, 1), lambda i: (i, 0)),
            pl.BlockSpec(p0["w"].shape, lambda i: (0, 0)),
            pl.BlockSpec(gb1.shape, lambda i: (0, 0)),
            pl.BlockSpec(p1["w"].shape, lambda i: (0, 0)),
            pl.BlockSpec(gb2.shape, lambda i: (0, 0)),
        ],
        out_specs=pl.BlockSpec((G, G), lambda i: (0, 0)),
        out_shape=jax.ShapeDtypeStruct((G, G), F32),
        scratch_shapes=[pltpu.VMEM((G, CH), F32)],
    )(xn_out, b2d, p0["w"], gb1, p1["w"], gb2)


NC, NS = 2, 16
NW = NC * NS

GCHUNK = 40
G_PER_W = 2 * EH // NW
G_NCH = G_PER_W // GCHUNK

S_PER_W = EH // NS
S_NCH = S_PER_W // GCHUNK
CHH = CH // 2


def _gather_rows(table, idx_r):
    mesh = plsc.VectorSubcoreMesh(core_axis_name="c", subcore_axis_name="s")

    @functools.partial(
        pl.kernel,
        out_type=jax.ShapeDtypeStruct((2 * EH, CHH), jnp.int32),
        mesh=mesh,
        scratch_types=[
            pltpu.VMEM((G_NCH, GCHUNK), jnp.int32),
            pltpu.VMEM((GCHUNK, CHH), jnp.int32),
            pltpu.VMEM((GCHUNK, CHH), jnp.int32),
            pltpu.SemaphoreType.DMA,
            pltpu.SemaphoreType.DMA,
        ],
    )
    def k(table_hbm, idx_hbm, out_hbm, idx_v, buf0, buf1, sem0, sem1):
        wid = lax.axis_index("s") * NC + lax.axis_index("c")
        base = wid * G_PER_W
        pltpu.sync_copy(idx_hbm.at[wid], idx_v)
        pltpu.async_copy(table_hbm.at[idx_v.at[0]], buf0, sem0)

        def body(t, _):
            a = 2 * t

            @pl.when(a + 1 < G_NCH)
            def _l1():
                pltpu.async_copy(table_hbm.at[idx_v.at[a + 1]], buf1, sem1)

            pltpu.make_async_copy(table_hbm.at[idx_v.at[a]], buf0, sem0).wait()
            pltpu.sync_copy(buf0,
                            out_hbm.at[pl.ds(base + a * GCHUNK, GCHUNK)])

            @pl.when(a + 2 < G_NCH)
            def _l2():
                pltpu.async_copy(table_hbm.at[idx_v.at[a + 2]], buf0, sem0)

            @pl.when(a + 1 < G_NCH)
            def _w1():
                pltpu.make_async_copy(table_hbm.at[idx_v.at[a + 1]], buf1,
                                      sem1).wait()
                pltpu.sync_copy(
                    buf1, out_hbm.at[pl.ds(base + (a + 1) * GCHUNK, GCHUNK)])

            return 0

        lax.fori_loop(0, (G_NCH + 1) // 2, body, 0, unroll=False)

    return k(table, idx_r)


def _segment_sum(xe_h, idx_r, zeros_half):
    mesh = plsc.VectorSubcoreMesh(core_axis_name="c", subcore_axis_name="s")

    @functools.partial(
        pl.kernel,
        out_type=jax.ShapeDtypeStruct((N, CH), F32),
        mesh=mesh,
        scratch_types=[
            pltpu.VMEM((S_NCH, GCHUNK), jnp.int32),
            pltpu.VMEM((GCHUNK, CHH), F32),
            pltpu.VMEM((GCHUNK, CHH), F32),
            pltpu.VMEM_SHARED((N, CHH), F32),
            pltpu.SemaphoreType.DMA,
            pltpu.SemaphoreType.DMA,
        ],
    )
    def k(xe_hbm, idx_hbm, z_hbm, out_hbm, idx_v, buf0, buf1, acc, sem0,
          sem1):
        cid = lax.axis_index("c")
        sid = lax.axis_index("s")
        col0 = cid * CHH
        zbase = sid * 624
        pltpu.sync_copy(z_hbm.at[pl.ds(zbase, 624)], acc.at[pl.ds(zbase, 624)])

        @pl.when(sid == NS - 1)
        def _tail():
            pltpu.sync_copy(z_hbm.at[pl.ds(9984, 16)], acc.at[pl.ds(9984, 16)])

        plsc.subcore_barrier()

        ebase = sid * S_PER_W
        pltpu.sync_copy(idx_hbm.at[sid], idx_v)

        def _src(j):
            return xe_hbm.at[pl.ds(ebase + j * GCHUNK, GCHUNK),
                             pl.ds(col0, CHH)]

        pltpu.async_copy(_src(0), buf0, sem0)

        def body(t, _):
            a = 2 * t

            @pl.when(a + 1 < S_NCH)
            def _l1():
                pltpu.async_copy(_src(a + 1), buf1, sem1)

            pltpu.make_async_copy(_src(a), buf0, sem0).wait()
            pltpu.sync_copy(buf0, acc.at[idx_v.at[a]], add=True)

            @pl.when(a + 2 < S_NCH)
            def _l2():
                pltpu.async_copy(_src(a + 2), buf0, sem0)

            @pl.when(a + 1 < S_NCH)
            def _w1():
                pltpu.make_async_copy(_src(a + 1), buf1, sem1).wait()
                pltpu.sync_copy(buf1, acc.at[idx_v.at[a + 1]], add=True)

            return 0

        lax.fori_loop(0, (S_NCH + 1) // 2, body, 0, unroll=False)
        plsc.subcore_barrier()
        pltpu.sync_copy(acc.at[pl.ds(zbase, 624)],
                        out_hbm.at[pl.ds(zbase, 624), pl.ds(col0, CHH)])

        @pl.when(sid == NS - 1)
        def _tail2():
            pltpu.sync_copy(acc.at[pl.ds(9984, 16)],
                            out_hbm.at[pl.ds(9984, 16), pl.ds(col0, CHH)])

    return k(xe_h, idx_r, zeros_half)


def kernel(x_nodes, x_edges, params, edge_index, batch, pbc):
    sender = edge_index[0]
    receiver = edge_index[1]

    idx2 = []
    recv_r = []
    for h in range(NHALF):
        snd_h = lax.dynamic_slice_in_dim(sender, h * EH, EH)
        rcv_h = lax.dynamic_slice_in_dim(receiver, h * EH, EH)
        idx2.append(jnp.concatenate([snd_h, rcv_h + N])
                    .reshape(NW, G_NCH, GCHUNK))
        recv_r.append(rcv_h.reshape(NS, S_NCH, GCHUNK))
    zeros_half = jnp.zeros((N, CHH), F32)

    nbh = EH // BE
    xe = [_mlp2(x_edges, *params["embed_edges"], block=BE, nb=nbh,
                off=h * nbh) for h in range(NHALF)]
    xn = _mlp2(x_nodes, *params["embed_nodes"], block=BN, nb=N // BN)

    for lp in params["layers"]:
        w1 = lp["edge"][0]["w"]
        wsr = jnp.stack([w1[:CH], w1[CH:2 * CH]])
        we = w1[2 * CH:]
        b1 = lp["edge"][0]["b"]
        w2, b2 = lp["edge"][1]["w"], lp["edge"][1]["b"]

        T = _pq(xn, wsr)
        gath = [_gather_rows(T, idx2[h]) for h in range(NHALF)]
        xe = [_edge_layer(gath[h], xe[h], we, b1, w2, b2)
              for h in range(NHALF)]
        aggr = [_segment_sum(xe[h], recv_r[h], zeros_half)
                for h in range(NHALF)]

        nw1 = lp["node"][0]["w"]
        xn = _node_layer(xn, aggr[0], aggr[1], nw1[:CH], nw1[CH:],
                         lp["node"][0]["b"], lp["node"][1]["w"],
                         lp["node"][1]["b"])

    xn_out = _mlp2(xn, *params["node_readout"], block=BN, nb=N // BN)
    xe_out = jnp.concatenate(
        [_mlp2(xe[h], *params["edge_readout"], block=BE, nb=nbh)
         for h in range(NHALF)])
    xg = _global_readout(xn_out, batch, *params["global_readout"])
    return (xn_out, xe_out, xg)
